# baseline (device time: 141643 ns/iter reference)
import jax
import jax.numpy as jnp
from jax import lax
from jax.experimental import pallas as pl
from jax.experimental.pallas import tpu as pltpu

N_DEV = 8
_GELU_C = 0.7978845608028654

_NSLOT = 6


def kernel(x, w_mat):
    m_per, k_dim = x.shape
    _, n_full = w_mat.shape
    n_per = n_full // N_DEV
    kc = 1024
    n_wc = k_dim // kc

    tasks = []
    for c in range(n_wc):
        tasks.append(("x", 2 * c))
        tasks.append(("x", 2 * c + 1))
        tasks.append(("w", 0, c))
    for j in range(1, N_DEV):
        for c in range(n_wc):
            tasks.append(("w", j, c))
    n_task = len(tasks)

    def body(x_hbm, w_hbm, out_hbm, x_bf, stage, out_stage,
             send_buf, recv_buf, stage_sems, out_sems, send_sems, recv_sems):
        my = lax.axis_index("i")

        barrier = pltpu.get_barrier_semaphore()
        for d in range(1, N_DEV):
            pl.semaphore_signal(
                barrier, inc=1,
                device_id=((my + d) % N_DEV,),
                device_id_type=pl.DeviceIdType.MESH,
            )
        pl.semaphore_wait(barrier, N_DEV - 1)

        def task_dma(t):
            slot = t % _NSLOT
            task = tasks[t]
            if task[0] == "x":
                src = x_hbm.at[:, pl.ds(task[1] * n_per, n_per)]
            else:
                _, j, c = task
                dst = (my + j) % N_DEV
                src = w_hbm.at[pl.ds(c * kc, kc), pl.ds(dst * n_per, n_per)]
            return pltpu.make_async_copy(src, stage.at[slot], stage_sems.at[slot])

        def hop_rdma(j):
            return pltpu.make_async_remote_copy(
                src_ref=send_buf.at[j % 2],
                dst_ref=recv_buf.at[j],
                send_sem=send_sems.at[j],
                recv_sem=recv_sems.at[j],
                device_id=((my + j) % N_DEV,),
                device_id_type=pl.DeviceIdType.MESH,
            )

        out_block = [0]

        def emit_out(row0, val_f32):
            b = out_block[0]
            out_block[0] += 1
            s = b % 2
            if b >= 2:
                pltpu.make_async_copy(
                    out_stage.at[s], out_hbm.at[pl.ds(0, m_per), :],
                    out_sems.at[s],
                ).wait()
            out_stage[s] = val_f32
            pltpu.make_async_copy(
                out_stage.at[s], out_hbm.at[pl.ds(row0, m_per), :],
                out_sems.at[s],
            ).start()

        def drain(j):
            src = (my - j) % N_DEV
            hop_rdma(j).wait_recv()
            emit_out(src * m_per, recv_buf[j].astype(jnp.float32))

        def finish_step(j, y):
            g = 0.5 * y * (1.0 + jnp.tanh(_GELU_C * (y + 0.044715 * y * y * y)))
            if j == 0:
                emit_out(my * m_per, g)
            else:
                if j >= 3:
                    hop_rdma(j - 2).wait_send()
                send_buf[j % 2] = g.astype(jnp.bfloat16)
                hop_rdma(j).start()
            if j >= 2:
                drain(j - 1)

        for t in range(_NSLOT):
            task_dma(t).start()

        y = None
        for t, task in enumerate(tasks):
            task_dma(t).wait()
            v = stage[t % _NSLOT].astype(jnp.bfloat16)
            if task[0] == "x":
                x_bf[:, pl.ds(task[1] * n_per, n_per)] = v
            else:
                _, j, c = task
                part = jnp.dot(x_bf[:, pl.ds(c * kc, kc)], v,
                               preferred_element_type=jnp.float32)
                y = part if y is None else y + part
            if t + _NSLOT < n_task:
                task_dma(t + _NSLOT).start()
            if task[0] == "w" and task[2] == n_wc - 1:
                finish_step(task[1], y)
                y = None

        drain(N_DEV - 1)

        for b in (out_block[0] - 2, out_block[0] - 1):
            pltpu.make_async_copy(
                out_stage.at[b % 2], out_hbm.at[pl.ds(0, m_per), :],
                out_sems.at[b % 2],
            ).wait()
        hop_rdma(N_DEV - 2).wait_send()
        hop_rdma(N_DEV - 1).wait_send()

    return pl.pallas_call(
        body,
        out_shape=jax.ShapeDtypeStruct((N_DEV * m_per, n_per), jnp.float32),
        in_specs=[
            pl.BlockSpec(memory_space=pl.ANY),
            pl.BlockSpec(memory_space=pl.ANY),
        ],
        out_specs=pl.BlockSpec(memory_space=pl.ANY),
        scratch_shapes=[
            pltpu.VMEM((m_per, k_dim), jnp.bfloat16),
            pltpu.VMEM((_NSLOT, m_per, n_per), jnp.float32),
            pltpu.VMEM((2, m_per, n_per), jnp.float32),
            pltpu.VMEM((2, m_per, n_per), jnp.bfloat16),
            pltpu.VMEM((N_DEV, m_per, n_per), jnp.bfloat16),
            pltpu.SemaphoreType.DMA((_NSLOT,)),
            pltpu.SemaphoreType.DMA((2,)),
            pltpu.SemaphoreType.DMA((N_DEV,)),
            pltpu.SemaphoreType.DMA((N_DEV,)),
        ],
        compiler_params=pltpu.CompilerParams(
            collective_id=0,
            vmem_limit_bytes=63 * 1024 * 1024,
        ),
    )(x, w_mat)


# device time: 138759 ns/iter; 1.0208x vs baseline; 1.0208x over previous
import os

import jax
import jax.numpy as jnp
from jax import lax
from jax.experimental import pallas as pl
from jax.experimental.pallas import tpu as pltpu

_VARIANT = os.environ.get("KERNEL_VARIANT", "full")

N_DEV = 8
_GELU_C = 0.7978845608028654

_NSLOT = 6


def kernel(x, w_mat):
    m_per, k_dim = x.shape
    _, n_full = w_mat.shape
    n_per = n_full // N_DEV
    kc = 1024
    n_wc = k_dim // kc
    n_xc = k_dim // n_per
    xpw = n_xc // n_wc

    tasks = []
    for c in range(n_wc):
        for i in range(xpw):
            tasks.append(("x", xpw * c + i))
        tasks.append(("w", 0, c))
    for j in range(1, N_DEV):
        for c in range(n_wc):
            tasks.append(("w", j, c))
    n_task = len(tasks)

    def body(x_hbm, w_hbm, out_hbm, x_bf, stage, out_stage,
             send_buf, recv_buf, stage_sems, out_sems, send_sems, recv_sems):
        my = lax.axis_index("i")

        barrier = pltpu.get_barrier_semaphore()
        for d in range(1, N_DEV):
            pl.semaphore_signal(
                barrier, inc=1,
                device_id=((my + d) % N_DEV,),
                device_id_type=pl.DeviceIdType.MESH,
            )
        pl.semaphore_wait(barrier, N_DEV - 1)

        def task_dma(t):
            slot = t % _NSLOT
            task = tasks[t]
            if task[0] == "x":
                src = x_hbm.at[:, pl.ds(task[1] * n_per, n_per)]
                dst_ref = stage.at[slot, pl.ds(0, m_per), :]
            else:
                _, j, c = task
                dst = (my + j) % N_DEV
                src = w_hbm.at[pl.ds(c * kc, kc), pl.ds(dst * n_per, n_per)]
                dst_ref = stage.at[slot]
            return pltpu.make_async_copy(src, dst_ref, stage_sems.at[slot])

        def hop_rdma(j):
            return pltpu.make_async_remote_copy(
                src_ref=send_buf.at[j % 2],
                dst_ref=recv_buf.at[j],
                send_sem=send_sems.at[j],
                recv_sem=recv_sems.at[j],
                device_id=((my + j) % N_DEV,),
                device_id_type=pl.DeviceIdType.MESH,
            )

        out_block = [0]

        def emit_out(row0, val_f32):
            b = out_block[0]
            out_block[0] += 1
            s = b % 2
            if b >= 2:
                pltpu.make_async_copy(
                    out_stage.at[s], out_hbm.at[pl.ds(0, m_per), :],
                    out_sems.at[s],
                ).wait()
            out_stage[s] = val_f32
            pltpu.make_async_copy(
                out_stage.at[s], out_hbm.at[pl.ds(row0, m_per), :],
                out_sems.at[s],
            ).start()

        def drain(j):
            src = (my - j) % N_DEV
            hop_rdma(j).wait_recv()
            emit_out(src * m_per, recv_buf[j].astype(jnp.float32))

        def finish_step(j, y):
            g = 0.5 * y * (1.0 + jnp.tanh(_GELU_C * (y + 0.044715 * y * y * y)))
            if j == 0:
                emit_out(my * m_per, g)
            else:
                if j >= 3:
                    hop_rdma(j - 2).wait_send()
                send_buf[j % 2] = g.astype(jnp.bfloat16)
                hop_rdma(j).start()
            if j >= 2:
                drain(j - 1)

        for t in range(_NSLOT):
            task_dma(t).start()

        y = None
        for t, task in enumerate(tasks):
            task_dma(t).wait()
            if task[0] == "x":
                x_bf[:, pl.ds(task[1] * n_per, n_per)] = (
                    stage[t % _NSLOT, pl.ds(0, m_per), :].astype(jnp.bfloat16))
            else:
                _, j, c = task
                if _VARIANT == "nodot":
                    send_buf[0] = stage[t % _NSLOT, pl.ds(0, m_per), :].astype(
                        jnp.bfloat16)
                else:
                    if _VARIANT == "nocast":
                        v = x_bf[:, pl.ds(0, n_per)]
                    else:
                        v = stage[t % _NSLOT].astype(jnp.bfloat16)
                    part = jnp.dot(x_bf[:, pl.ds(c * kc, kc)], v,
                                   preferred_element_type=jnp.float32)
                    y = part if y is None else y + part
            if t + _NSLOT < n_task:
                task_dma(t + _NSLOT).start()
            if task[0] == "w" and task[2] == n_wc - 1 and _VARIANT != "nodot":
                finish_step(task[1], y)
                y = None

        if _VARIANT != "nodot":
            drain(N_DEV - 1)

        if _VARIANT != "nodot":
            for b in (out_block[0] - 2, out_block[0] - 1):
                pltpu.make_async_copy(
                    out_stage.at[b % 2], out_hbm.at[pl.ds(0, m_per), :],
                    out_sems.at[b % 2],
                ).wait()
            hop_rdma(N_DEV - 2).wait_send()
            hop_rdma(N_DEV - 1).wait_send()

    return pl.pallas_call(
        body,
        out_shape=jax.ShapeDtypeStruct((N_DEV * m_per, n_per), jnp.float32),
        in_specs=[
            pl.BlockSpec(memory_space=pl.ANY),
            pl.BlockSpec(memory_space=pl.ANY),
        ],
        out_specs=pl.BlockSpec(memory_space=pl.ANY),
        scratch_shapes=[
            pltpu.VMEM((m_per, k_dim), jnp.bfloat16),
            pltpu.VMEM((_NSLOT, kc, n_per), jnp.float32),
            pltpu.VMEM((2, m_per, n_per), jnp.float32),
            pltpu.VMEM((2, m_per, n_per), jnp.bfloat16),
            pltpu.VMEM((N_DEV, m_per, n_per), jnp.bfloat16),
            pltpu.SemaphoreType.DMA((_NSLOT,)),
            pltpu.SemaphoreType.DMA((2,)),
            pltpu.SemaphoreType.DMA((N_DEV,)),
            pltpu.SemaphoreType.DMA((N_DEV,)),
        ],
        compiler_params=pltpu.CompilerParams(
            collective_id=0,
            vmem_limit_bytes=63 * 1024 * 1024,
        ),
    )(x, w_mat)
